# baseline (speedup 1.0000x reference)
import functools
import math

import jax
import jax.numpy as jnp
from jax import lax
from jax.experimental import pallas as pl
from jax.experimental.pallas import tpu as pltpu
from jax.experimental.pallas import tpu_sc as plsc

_NUM_CT = 5
_TEMP = 20.0
_GAMMA1 = 0.8
_TAU = 0.1
_BETA = 1.0
_EPS = float(jnp.finfo(jnp.float32).eps)
_INV_T = 1.0 / _TEMP
_INV_TAU = 1.0 / _TAU
_LOG2E = math.log2(math.e)
_C1 = _INV_T * _LOG2E
_C2 = _INV_TAU * _LOG2E

_BI = 256


def _main_body(x_ref, y_ref, xc_ref, tc_ref, slr_ref, slf_ref, labf_ref,
               gi_ref, gt_ref, ep_ref,
               out_ref, va_s, vb_s, ce_s, c0_s, d0_s, dr_s):
    pid = pl.program_id(0)
    nb = pl.num_programs(0)
    bi, b = x_ref.shape[0], y_ref.shape[0]
    i0 = pid * bi

    @pl.when(pid == 0)
    def _init():
        c0_s[...] = jnp.zeros_like(c0_s)
        d0_s[...] = jnp.zeros_like(d0_s)

    sim = lax.dot_general(x_ref[...], y_ref[...], (((1,), (1,)), ((), ())),
                          preferred_element_type=jnp.float32)
    f = jnp.exp2(sim * _C1)

    d_b = jnp.sum(x_ref[...] * y_ref[pl.ds(i0, bi), :], axis=1,
                  keepdims=True)
    dmask = (lax.broadcasted_iota(jnp.int32, (bi, bi), 0)
             == lax.broadcasted_iota(jnp.int32, (bi, bi), 1)).astype(
                 jnp.float32)
    for k in range(b // bi):
        @pl.when(pid == k)
        def _diag_k(k=k):
            dr_s[0:1, k * bi:(k + 1) * bi] = jnp.sum(
                sim[:, k * bi:(k + 1) * bi] * dmask, axis=0, keepdims=True)

    neg_row = (slr_ref[...] != 1).astype(jnp.float32)
    neg_col = (slf_ref[pl.ds(i0, bi), :] != 1).astype(jnp.float32)

    fnc = f * neg_col
    c0_s[...] += jnp.sum(fnc, axis=0, keepdims=True)
    d0_s[...] += jnp.sum(fnc * sim, axis=0, keepdims=True)

    fnr = f * neg_row
    row_f = jnp.sum(fnr, axis=1, keepdims=True)
    row_fs = jnp.sum(fnr * sim, axis=1, keepdims=True)
    esc = jnp.exp2(-d_b * _C1)
    a = esc * row_f
    va_s[pl.ds(i0, bi), :] = a
    vb_s[pl.ds(i0, bi), :] = esc * row_fs - d_b * a

    raw = lax.dot_general(xc_ref[...], tc_ref[...], (((1,), (1,)), ((), ())),
                          preferred_element_type=jnp.float32)
    m = jnp.max(raw, axis=1, keepdims=True)
    z = jnp.sum(jnp.exp2((raw - m) * _C2), axis=1, keepdims=True)
    lse = _INV_TAU * m + jnp.log(z)
    rsub = raw[:, 0:128]
    col = lax.broadcasted_iota(jnp.int32, (bi, 128), 1)
    lab_b = labf_ref[pl.ds(i0, bi), :]
    picked = jnp.sum(jnp.where(col == lab_b, rsub, 0.0),
                     axis=1, keepdims=True)
    ce_s[pl.ds(i0, bi), :] = lse - _INV_TAU * picked

    @pl.when(pid == nb - 1)
    def _fin():
        ep = ep_ref[0, 0]
        pos_row = (slr_ref[...] == 1).astype(jnp.float32)
        n_pos = jnp.sum(pos_row)
        n_neg = jnp.sum(1.0 - pos_row)

        posf = (slf_ref[...] == 1).astype(jnp.float32)
        g_i = va_s[...] / n_neg
        s_i = jnp.where(ep == 0, g_i,
                        (1.0 - _GAMMA1) * gi_ref[...] + _GAMMA1 * g_i)
        img_sum = jnp.sum(posf * vb_s[...] / (s_i + _EPS), keepdims=True)

        d_row = dr_s[...]
        scale = jnp.exp2(-d_row * _C1)
        c_v = scale * c0_s[...]
        dv = scale * (d0_s[...] - d_row * c0_s[...])
        g_t = c_v / n_neg
        s_t = jnp.where(ep == 0, g_t,
                        (1.0 - _GAMMA1) * gt_ref[...] + _GAMMA1 * g_t)
        text_sum = jnp.sum(pos_row * dv / (s_t + _EPS), keepdims=True)

        contrast = (img_sum + text_sum) / (n_neg * n_pos)

        ce = ce_s[...]
        lab = labf_ref[...]
        total = jnp.zeros((1, 1), jnp.float32)
        npres = jnp.zeros((1, 1), jnp.float32)
        for c in range(_NUM_CT):
            mc = (lab == c).astype(jnp.float32)
            nc = jnp.sum(mc)
            sc = jnp.sum(mc * ce, keepdims=True)
            pres = (nc > 0).astype(jnp.float32)
            total += pres * sc / jnp.maximum(nc, 1.0)
            npres += pres
        out_ref[...] = contrast + _BETA * _TAU * total / npres


def _main(x, y, xc, tc, slabel, labels, g_i, g_t, epoch_arr, interpret=False):
    b, d = x.shape
    nb = b // _BI
    return pl.pallas_call(
        _main_body,
        grid=(nb,),
        in_specs=[
            pl.BlockSpec((_BI, d), lambda i: (i, 0)),
            pl.BlockSpec((b, d), lambda i: (0, 0)),
            pl.BlockSpec((_BI, d), lambda i: (i, 0)),
            pl.BlockSpec((b, d), lambda i: (0, 0)),
            pl.BlockSpec((1, b), lambda i: (0, 0)),
            pl.BlockSpec((b, 1), lambda i: (0, 0)),
            pl.BlockSpec((b, 1), lambda i: (0, 0)),
            pl.BlockSpec((b, 1), lambda i: (0, 0)),
            pl.BlockSpec((1, b), lambda i: (0, 0)),
            pl.BlockSpec(memory_space=pltpu.SMEM),
        ],
        out_specs=pl.BlockSpec((1, 1), lambda i: (0, 0)),
        out_shape=jax.ShapeDtypeStruct((1, 1), jnp.float32),
        scratch_shapes=[
            pltpu.VMEM((b, 1), jnp.float32),
            pltpu.VMEM((b, 1), jnp.float32),
            pltpu.VMEM((b, 1), jnp.float32),
            pltpu.VMEM((1, b), jnp.float32),
            pltpu.VMEM((1, b), jnp.float32),
            pltpu.VMEM((1, b), jnp.float32),
        ],
        compiler_params=pltpu.CompilerParams(
            dimension_semantics=("arbitrary",)),
        interpret=interpret,
    )(x, y, xc, tc, slabel.reshape(1, b), slabel.reshape(b, 1),
      labels.reshape(b, 1), g_i.reshape(b, 1), g_t.reshape(1, b), epoch_arr)


def _gather_moments(s_i, image_ids, s_t, text_ids):
    b = image_ids.shape[0]
    info = plsc.get_sparse_core_info()
    nw = info.num_cores * info.num_subcores
    b_per_w = b // nw
    mesh = plsc.VectorSubcoreMesh(core_axis_name="c", subcore_axis_name="s")

    @functools.partial(
        pl.kernel, mesh=mesh,
        out_type=(jax.ShapeDtypeStruct((b,), jnp.float32),
                  jax.ShapeDtypeStruct((b,), jnp.float32)),
        scratch_types=[
            pltpu.VMEM((b_per_w,), jnp.int32),
            pltpu.VMEM((b_per_w,), jnp.float32),
            pltpu.VMEM((b_per_w,), jnp.int32),
            pltpu.VMEM((b_per_w,), jnp.float32),
            pltpu.SemaphoreType.DMA,
            pltpu.SemaphoreType.DMA,
        ],
    )
    def gk(t1_hbm, i1_hbm, t2_hbm, i2_hbm, o1_hbm, o2_hbm,
           idx1_v, row1_v, idx2_v, row2_v, sem1, sem2):
        wid = lax.axis_index("s") * info.num_cores + lax.axis_index("c")
        base = wid * b_per_w
        pltpu.sync_copy(i1_hbm.at[pl.ds(base, b_per_w)], idx1_v)
        pltpu.sync_copy(i2_hbm.at[pl.ds(base, b_per_w)], idx2_v)
        cp1 = pltpu.async_copy(t1_hbm.at[idx1_v], row1_v, sem1)
        cp2 = pltpu.async_copy(t2_hbm.at[idx2_v], row2_v, sem2)
        cp1.wait()
        cp2.wait()
        pltpu.sync_copy(row1_v, o1_hbm.at[pl.ds(base, b_per_w)])
        pltpu.sync_copy(row2_v, o2_hbm.at[pl.ds(base, b_per_w)])

    return gk(s_i, image_ids.astype(jnp.int32), s_t, text_ids.astype(jnp.int32))


def kernel(image_features, text_features, image_ids, text_ids, slabel, epoch,
           img_feas_c, txt_feas_c, labels_c, index_c, s_I, s_T):
    g_i, g_t = _gather_moments(s_I, image_ids, s_T, text_ids)
    epoch_arr = jnp.asarray(epoch, jnp.int32).reshape(1, 1)
    out = _main(image_features, text_features, img_feas_c, txt_feas_c,
                slabel.astype(jnp.int32), labels_c.astype(jnp.int32),
                g_i, g_t, epoch_arr)
    return out[0, 0]


# speedup vs baseline: 1.2088x; 1.2088x over previous
import functools
import math

import jax
import jax.numpy as jnp
from jax import lax
from jax.experimental import pallas as pl
from jax.experimental.pallas import tpu as pltpu
from jax.experimental.pallas import tpu_sc as plsc

_NUM_CT = 5
_TEMP = 20.0
_GAMMA1 = 0.8
_TAU = 0.1
_BETA = 1.0
_EPS = float(jnp.finfo(jnp.float32).eps)
_INV_T = 1.0 / _TEMP
_INV_TAU = 1.0 / _TAU
_LOG2E = math.log2(math.e)
_C1 = _INV_T * _LOG2E
_C2 = _INV_TAU * _LOG2E

_BI = 1024


def _main_body(x_ref, y_ref, xc_ref, tc_ref, slc_ref, slr_ref, labb_ref,
               va_ref, vb_ref, ce_ref, c0_ref, d0_ref, dc_ref):
    pid = pl.program_id(0)
    bi, b = x_ref.shape[0], y_ref.shape[0]
    i0 = pid * bi

    @pl.when(pid == 0)
    def _init():
        c0_ref[...] = jnp.zeros_like(c0_ref)
        d0_ref[...] = jnp.zeros_like(d0_ref)

    xb = x_ref[...]
    sim = lax.dot_general(xb, y_ref[...], (((1,), (1,)), ((), ())),
                          preferred_element_type=jnp.float32)
    f = jnp.exp2(sim * _C1)

    d_b = jnp.sum(xb * y_ref[pl.ds(i0, bi), :], axis=1, keepdims=True)
    dc_ref[...] = d_b

    neg_row = (slr_ref[...] != 1).astype(jnp.float32)
    neg_col = (slc_ref[...] != 1).astype(jnp.float32)

    fnc = f * neg_col
    c0_ref[...] += jnp.sum(fnc, axis=0, keepdims=True)
    d0_ref[...] += jnp.sum(fnc * sim, axis=0, keepdims=True)

    fnr = f * neg_row
    row_f = jnp.sum(fnr, axis=1, keepdims=True)
    row_fs = jnp.sum(fnr * sim, axis=1, keepdims=True)
    esc = jnp.exp2(-d_b * _C1)
    a = esc * row_f
    va_ref[...] = a
    vb_ref[...] = esc * row_fs - d_b * a

    raw = lax.dot_general(xc_ref[...], tc_ref[...], (((1,), (1,)), ((), ())),
                          preferred_element_type=jnp.float32)
    m = jnp.max(raw, axis=1, keepdims=True)
    z = jnp.sum(jnp.exp2((raw - m) * _C2), axis=1, keepdims=True)
    lse = _INV_TAU * m + jnp.log(z)
    rsub = raw[:, 0:128]
    col = lax.broadcasted_iota(jnp.int32, (bi, 128), 1)
    picked = jnp.sum(jnp.where(col == labb_ref[...], rsub, 0.0),
                     axis=1, keepdims=True)
    ce_ref[...] = lse - _INV_TAU * picked


def _main(x, y, xc, tc, slabel, labels, interpret=False):
    b, d = x.shape
    nb = b // _BI
    return pl.pallas_call(
        _main_body,
        grid=(nb,),
        in_specs=[
            pl.BlockSpec((_BI, d), lambda i: (i, 0)),
            pl.BlockSpec((b, d), lambda i: (0, 0)),
            pl.BlockSpec((_BI, d), lambda i: (i, 0)),
            pl.BlockSpec((b, d), lambda i: (0, 0)),
            pl.BlockSpec((_BI, 1), lambda i: (i, 0)),
            pl.BlockSpec((1, b), lambda i: (0, 0)),
            pl.BlockSpec((_BI, 1), lambda i: (i, 0)),
        ],
        out_specs=[
            pl.BlockSpec((_BI, 1), lambda i: (i, 0)),
            pl.BlockSpec((_BI, 1), lambda i: (i, 0)),
            pl.BlockSpec((_BI, 1), lambda i: (i, 0)),
            pl.BlockSpec((1, b), lambda i: (0, 0)),
            pl.BlockSpec((1, b), lambda i: (0, 0)),
            pl.BlockSpec((_BI, 1), lambda i: (i, 0)),
        ],
        out_shape=[
            jax.ShapeDtypeStruct((b, 1), jnp.float32),
            jax.ShapeDtypeStruct((b, 1), jnp.float32),
            jax.ShapeDtypeStruct((b, 1), jnp.float32),
            jax.ShapeDtypeStruct((1, b), jnp.float32),
            jax.ShapeDtypeStruct((1, b), jnp.float32),
            jax.ShapeDtypeStruct((b, 1), jnp.float32),
        ],
        compiler_params=pltpu.CompilerParams(
            dimension_semantics=("arbitrary",)),
        interpret=interpret,
    )(x, y, xc, tc, slabel.reshape(b, 1), slabel.reshape(1, b),
      labels.reshape(b, 1))


def _fin_body(va_ref, vb_ref, ce_ref, c0_ref, d0_ref, d_ref, sl_ref, lab_ref,
              gi_ref, gt_ref, ep_ref, out_ref):
    ep = ep_ref[0, 0]
    sl = sl_ref[...]
    pos = (sl == 1).astype(jnp.float32)
    neg = 1.0 - pos
    n_pos = jnp.sum(pos)
    n_neg = jnp.sum(neg)

    g_i = va_ref[...] / n_neg
    s_i = jnp.where(ep == 0, g_i, (1.0 - _GAMMA1) * gi_ref[...] + _GAMMA1 * g_i)
    img_sum = jnp.sum(pos * vb_ref[...] / (s_i + _EPS), keepdims=True)

    d_v = d_ref[...]
    scale = jnp.exp2(-d_v * _C1)
    c_v = scale * c0_ref[...]
    dv = scale * (d0_ref[...] - d_v * c0_ref[...])
    g_t = c_v / n_neg
    s_t = jnp.where(ep == 0, g_t, (1.0 - _GAMMA1) * gt_ref[...] + _GAMMA1 * g_t)
    text_sum = jnp.sum(pos * dv / (s_t + _EPS), keepdims=True)

    contrast = (img_sum + text_sum) / (n_neg * n_pos)

    ce = ce_ref[...]
    lab = lab_ref[...]
    total = jnp.zeros((1, 1), jnp.float32)
    npres = jnp.zeros((1, 1), jnp.float32)
    for c in range(_NUM_CT):
        mc = (lab == c).astype(jnp.float32)
        nc = jnp.sum(mc)
        sc = jnp.sum(mc * ce, keepdims=True)
        pres = (nc > 0).astype(jnp.float32)
        total += pres * sc / jnp.maximum(nc, 1.0)
        npres += pres
    out_ref[...] = contrast + _BETA * _TAU * total / npres


def _finalize(va, vb, ce, c0, d0, d_col, slabel, labels, g_i, g_t, epoch_arr,
              interpret=False):
    b = slabel.shape[0]
    r = (32, b // 32)
    return pl.pallas_call(
        _fin_body,
        in_specs=[pl.BlockSpec(r, lambda: (0, 0))] * 10
        + [pl.BlockSpec(memory_space=pltpu.SMEM)],
        out_specs=pl.BlockSpec((1, 1), lambda: (0, 0)),
        out_shape=jax.ShapeDtypeStruct((1, 1), jnp.float32),
        interpret=interpret,
    )(va.reshape(r), vb.reshape(r), ce.reshape(r), c0.reshape(r),
      d0.reshape(r), d_col.reshape(r), slabel.reshape(r), labels.reshape(r),
      g_i.reshape(r), g_t.reshape(r), epoch_arr)


def _gather_moments(s_i, image_ids, s_t, text_ids):
    b = image_ids.shape[0]
    info = plsc.get_sparse_core_info()
    nw = info.num_cores * info.num_subcores
    b_per_w = b // nw
    mesh = plsc.VectorSubcoreMesh(core_axis_name="c", subcore_axis_name="s")

    @functools.partial(
        pl.kernel, mesh=mesh,
        out_type=(jax.ShapeDtypeStruct((b,), jnp.float32),
                  jax.ShapeDtypeStruct((b,), jnp.float32)),
        scratch_types=[
            pltpu.VMEM((b_per_w,), jnp.int32),
            pltpu.VMEM((b_per_w,), jnp.float32),
            pltpu.VMEM((b_per_w,), jnp.int32),
            pltpu.VMEM((b_per_w,), jnp.float32),
            pltpu.SemaphoreType.DMA,
            pltpu.SemaphoreType.DMA,
        ],
    )
    def gk(t1_hbm, i1_hbm, t2_hbm, i2_hbm, o1_hbm, o2_hbm,
           idx1_v, row1_v, idx2_v, row2_v, sem1, sem2):
        wid = lax.axis_index("s") * info.num_cores + lax.axis_index("c")
        base = wid * b_per_w
        pltpu.sync_copy(i1_hbm.at[pl.ds(base, b_per_w)], idx1_v)
        pltpu.sync_copy(i2_hbm.at[pl.ds(base, b_per_w)], idx2_v)
        cp1 = pltpu.async_copy(t1_hbm.at[idx1_v], row1_v, sem1)
        cp2 = pltpu.async_copy(t2_hbm.at[idx2_v], row2_v, sem2)
        cp1.wait()
        cp2.wait()
        pltpu.sync_copy(row1_v, o1_hbm.at[pl.ds(base, b_per_w)])
        pltpu.sync_copy(row2_v, o2_hbm.at[pl.ds(base, b_per_w)])

    return gk(s_i, image_ids.astype(jnp.int32), s_t, text_ids.astype(jnp.int32))


def kernel(image_features, text_features, image_ids, text_ids, slabel, epoch,
           img_feas_c, txt_feas_c, labels_c, index_c, s_I, s_T):
    epoch_arr = jnp.asarray(epoch, jnp.int32).reshape(1, 1)
    va, vb, ce, c0, d0, d_col = _main(image_features, text_features,
                                      img_feas_c, txt_feas_c,
                                      slabel.astype(jnp.int32),
                                      labels_c.astype(jnp.int32))
    g_i, g_t = _gather_moments(s_I, image_ids, s_T, text_ids)
    out = _finalize(va, vb, ce, c0, d0, d_col, slabel.astype(jnp.int32),
                    labels_c.astype(jnp.int32), g_i, g_t, epoch_arr)
    return out[0, 0]


# speedup vs baseline: 1.2126x; 1.0031x over previous
import functools
import math

import jax
import jax.numpy as jnp
from jax import lax
from jax.experimental import pallas as pl
from jax.experimental.pallas import tpu as pltpu
from jax.experimental.pallas import tpu_sc as plsc

_NUM_CT = 5
_TEMP = 20.0
_GAMMA1 = 0.8
_TAU = 0.1
_BETA = 1.0
_EPS = float(jnp.finfo(jnp.float32).eps)
_INV_T = 1.0 / _TEMP
_INV_TAU = 1.0 / _TAU
_LOG2E = math.log2(math.e)
_C1 = _INV_T * _LOG2E
_C2 = _INV_TAU * _LOG2E

_BI = 1024


def _main_body(x_ref, y_ref, xc_ref, tc_ref, slc_ref, slr_ref, labb_ref,
               va_ref, vb_ref, ce_ref, c0_ref, d0_ref, dc_ref,
               ybf_s, tcbf_s):
    pid = pl.program_id(0)
    bi, b = x_ref.shape[0], y_ref.shape[0]
    i0 = pid * bi

    @pl.when(pid == 0)
    def _init():
        c0_ref[...] = jnp.zeros_like(c0_ref)
        d0_ref[...] = jnp.zeros_like(d0_ref)
        ybf_s[...] = y_ref[...].astype(jnp.bfloat16)
        tcbf_s[...] = tc_ref[...].astype(jnp.bfloat16)

    xb = x_ref[...]
    sim = lax.dot_general(xb.astype(jnp.bfloat16), ybf_s[...],
                          (((1,), (1,)), ((), ())),
                          preferred_element_type=jnp.float32)
    f = jnp.exp2(sim * _C1)

    d_b = jnp.sum(xb * y_ref[pl.ds(i0, bi), :], axis=1, keepdims=True)
    dc_ref[...] = d_b

    neg_row = (slr_ref[...] != 1).astype(jnp.float32)
    neg_col = (slc_ref[...] != 1).astype(jnp.float32)

    fnc = f * neg_col
    c0_ref[...] += jnp.sum(fnc, axis=0, keepdims=True)
    d0_ref[...] += jnp.sum(fnc * sim, axis=0, keepdims=True)

    fnr = f * neg_row
    row_f = jnp.sum(fnr, axis=1, keepdims=True)
    row_fs = jnp.sum(fnr * sim, axis=1, keepdims=True)
    esc = jnp.exp2(-d_b * _C1)
    a = esc * row_f
    va_ref[...] = a
    vb_ref[...] = esc * row_fs - d_b * a

    raw = lax.dot_general(xc_ref[...].astype(jnp.bfloat16), tcbf_s[...],
                          (((1,), (1,)), ((), ())),
                          preferred_element_type=jnp.float32)
    m = jnp.max(raw, axis=1, keepdims=True)
    z = jnp.sum(jnp.exp2((raw - m) * _C2), axis=1, keepdims=True)
    lse = _INV_TAU * m + jnp.log(z)
    rsub = raw[:, 0:128]
    col = lax.broadcasted_iota(jnp.int32, (bi, 128), 1)
    picked = jnp.sum(jnp.where(col == labb_ref[...], rsub, 0.0),
                     axis=1, keepdims=True)
    ce_ref[...] = lse - _INV_TAU * picked


def _main(x, y, xc, tc, slabel, labels, interpret=False):
    b, d = x.shape
    nb = b // _BI
    return pl.pallas_call(
        _main_body,
        grid=(nb,),
        in_specs=[
            pl.BlockSpec((_BI, d), lambda i: (i, 0)),
            pl.BlockSpec((b, d), lambda i: (0, 0)),
            pl.BlockSpec((_BI, d), lambda i: (i, 0)),
            pl.BlockSpec((b, d), lambda i: (0, 0)),
            pl.BlockSpec((_BI, 1), lambda i: (i, 0)),
            pl.BlockSpec((1, b), lambda i: (0, 0)),
            pl.BlockSpec((_BI, 1), lambda i: (i, 0)),
        ],
        out_specs=[
            pl.BlockSpec((_BI, 1), lambda i: (i, 0)),
            pl.BlockSpec((_BI, 1), lambda i: (i, 0)),
            pl.BlockSpec((_BI, 1), lambda i: (i, 0)),
            pl.BlockSpec((1, b), lambda i: (0, 0)),
            pl.BlockSpec((1, b), lambda i: (0, 0)),
            pl.BlockSpec((_BI, 1), lambda i: (i, 0)),
        ],
        out_shape=[
            jax.ShapeDtypeStruct((b, 1), jnp.float32),
            jax.ShapeDtypeStruct((b, 1), jnp.float32),
            jax.ShapeDtypeStruct((b, 1), jnp.float32),
            jax.ShapeDtypeStruct((1, b), jnp.float32),
            jax.ShapeDtypeStruct((1, b), jnp.float32),
            jax.ShapeDtypeStruct((b, 1), jnp.float32),
        ],
        scratch_shapes=[
            pltpu.VMEM((b, d), jnp.bfloat16),
            pltpu.VMEM((b, d), jnp.bfloat16),
        ],
        compiler_params=pltpu.CompilerParams(
            dimension_semantics=("arbitrary",)),
        interpret=interpret,
    )(x, y, xc, tc, slabel.reshape(b, 1), slabel.reshape(1, b),
      labels.reshape(b, 1))


def _fin_body(va_ref, vb_ref, ce_ref, c0_ref, d0_ref, d_ref, sl_ref, lab_ref,
              gi_ref, gt_ref, ep_ref, out_ref):
    ep = ep_ref[0, 0]
    sl = sl_ref[...]
    pos = (sl == 1).astype(jnp.float32)
    neg = 1.0 - pos
    n_pos = jnp.sum(pos)
    n_neg = jnp.sum(neg)

    g_i = va_ref[...] / n_neg
    s_i = jnp.where(ep == 0, g_i, (1.0 - _GAMMA1) * gi_ref[...] + _GAMMA1 * g_i)
    img_sum = jnp.sum(pos * vb_ref[...] / (s_i + _EPS), keepdims=True)

    d_v = d_ref[...]
    scale = jnp.exp2(-d_v * _C1)
    c_v = scale * c0_ref[...]
    dv = scale * (d0_ref[...] - d_v * c0_ref[...])
    g_t = c_v / n_neg
    s_t = jnp.where(ep == 0, g_t, (1.0 - _GAMMA1) * gt_ref[...] + _GAMMA1 * g_t)
    text_sum = jnp.sum(pos * dv / (s_t + _EPS), keepdims=True)

    contrast = (img_sum + text_sum) / (n_neg * n_pos)

    ce = ce_ref[...]
    lab = lab_ref[...]
    total = jnp.zeros((1, 1), jnp.float32)
    npres = jnp.zeros((1, 1), jnp.float32)
    for c in range(_NUM_CT):
        mc = (lab == c).astype(jnp.float32)
        nc = jnp.sum(mc)
        sc = jnp.sum(mc * ce, keepdims=True)
        pres = (nc > 0).astype(jnp.float32)
        total += pres * sc / jnp.maximum(nc, 1.0)
        npres += pres
    out_ref[...] = contrast + _BETA * _TAU * total / npres


def _finalize(va, vb, ce, c0, d0, d_col, slabel, labels, g_i, g_t, epoch_arr,
              interpret=False):
    b = slabel.shape[0]
    r = (32, b // 32)
    return pl.pallas_call(
        _fin_body,
        in_specs=[pl.BlockSpec(r, lambda: (0, 0))] * 10
        + [pl.BlockSpec(memory_space=pltpu.SMEM)],
        out_specs=pl.BlockSpec((1, 1), lambda: (0, 0)),
        out_shape=jax.ShapeDtypeStruct((1, 1), jnp.float32),
        interpret=interpret,
    )(va.reshape(r), vb.reshape(r), ce.reshape(r), c0.reshape(r),
      d0.reshape(r), d_col.reshape(r), slabel.reshape(r), labels.reshape(r),
      g_i.reshape(r), g_t.reshape(r), epoch_arr)


def _gather_moments(s_i, image_ids, s_t, text_ids):
    b = image_ids.shape[0]
    info = plsc.get_sparse_core_info()
    nw = info.num_cores * info.num_subcores
    b_per_w = b // nw
    mesh = plsc.VectorSubcoreMesh(core_axis_name="c", subcore_axis_name="s")

    @functools.partial(
        pl.kernel, mesh=mesh,
        out_type=(jax.ShapeDtypeStruct((b,), jnp.float32),
                  jax.ShapeDtypeStruct((b,), jnp.float32)),
        scratch_types=[
            pltpu.VMEM((b_per_w,), jnp.int32),
            pltpu.VMEM((b_per_w,), jnp.float32),
            pltpu.VMEM((b_per_w,), jnp.int32),
            pltpu.VMEM((b_per_w,), jnp.float32),
            pltpu.SemaphoreType.DMA,
            pltpu.SemaphoreType.DMA,
        ],
    )
    def gk(t1_hbm, i1_hbm, t2_hbm, i2_hbm, o1_hbm, o2_hbm,
           idx1_v, row1_v, idx2_v, row2_v, sem1, sem2):
        wid = lax.axis_index("s") * info.num_cores + lax.axis_index("c")
        base = wid * b_per_w
        pltpu.sync_copy(i1_hbm.at[pl.ds(base, b_per_w)], idx1_v)
        pltpu.sync_copy(i2_hbm.at[pl.ds(base, b_per_w)], idx2_v)
        cp1 = pltpu.async_copy(t1_hbm.at[idx1_v], row1_v, sem1)
        cp2 = pltpu.async_copy(t2_hbm.at[idx2_v], row2_v, sem2)
        cp1.wait()
        cp2.wait()
        pltpu.sync_copy(row1_v, o1_hbm.at[pl.ds(base, b_per_w)])
        pltpu.sync_copy(row2_v, o2_hbm.at[pl.ds(base, b_per_w)])

    return gk(s_i, image_ids.astype(jnp.int32), s_t, text_ids.astype(jnp.int32))


def kernel(image_features, text_features, image_ids, text_ids, slabel, epoch,
           img_feas_c, txt_feas_c, labels_c, index_c, s_I, s_T):
    epoch_arr = jnp.asarray(epoch, jnp.int32).reshape(1, 1)
    va, vb, ce, c0, d0, d_col = _main(image_features, text_features,
                                      img_feas_c, txt_feas_c,
                                      slabel.astype(jnp.int32),
                                      labels_c.astype(jnp.int32))
    g_i, g_t = _gather_moments(s_I, image_ids, s_T, text_ids)
    out = _finalize(va, vb, ce, c0, d0, d_col, slabel.astype(jnp.int32),
                    labels_c.astype(jnp.int32), g_i, g_t, epoch_arr)
    return out[0, 0]
